# baseline (speedup 1.0000x reference)
import jax
import jax.numpy as jnp
from jax import lax
from jax.experimental import pallas as pl
from jax.experimental.pallas import tpu as pltpu

_MARGIN = 0.1
_ALPHA = 0.5
_BETA = 0.5
_C = 32
_N = 1024
_D = 2048
_NS = 4
_S = _N // _NS
_NEG_INF = float('-inf')
_POS_INF = float('inf')


def _nt(a, b):
    return lax.dot_general(a, b, (((1,), (1,)), ((), ())),
                           preferred_element_type=jnp.float32)


def _ntb(a, b):
    return lax.dot_general(a, b, (((1,), (1,)), ((), ())),
                           preferred_element_type=jnp.float32
                           ).astype(jnp.bfloat16)


def _block_updates(g, hb_col, hb_row, tc, tr, labels_row):
    v = jnp.where(tc == tr, hb_col + hb_row - g,
                  jnp.bfloat16(_NEG_INF))
    colmax = jnp.max(v, axis=0, keepdims=True).astype(jnp.float32)
    rowmax = jnp.max(v, axis=1, keepdims=True).astype(jnp.float32)
    rcls = jnp.max(jnp.where(tc == labels_row, rowmax, _NEG_INF),
                   axis=0, keepdims=True)
    return colmax, rcls


def _body(fs_ref, tcol_ref, trow_ref, out_ref,
          fall_scr, hb_scr, cen_scr, colmax_scr, rcls_scr):
    s = pl.program_id(0)
    fs = fs_ref[...]
    labels_row = lax.broadcasted_iota(jnp.int32, (1, _C), 1)

    ones_row = jnp.ones((1, _D), jnp.float32)
    ff = fs * fs
    hsq_s = 0.5 * _nt(ff, ones_row)
    hsqr_s = 0.5 * _nt(ones_row, ff)
    fb_s = fs.astype(jnp.bfloat16)
    hb_s = hsq_s.astype(jnp.bfloat16)
    hbr_s = hsqr_s.astype(jnp.bfloat16)

    for k in range(_NS):
        @pl.when(s == k)
        def _step(k=k):
            fall_scr[pl.ds(k * _S, _S), :] = fb_s
            hb_scr[:, pl.ds(k * _S, _S)] = hbr_s
            tc_k = tcol_ref[pl.ds(k * _S, _S), :]
            onehot_k = (tc_k == labels_row).astype(jnp.float32)
            cen_part = lax.dot_general(onehot_k, fs,
                                       (((0,), (0,)), ((), ())),
                                       preferred_element_type=jnp.float32)
            if k == 0:
                cen_scr[...] = cen_part
            else:
                cen_scr[...] += cen_part

            rcls_acc = None
            for j in range(k + 1):
                tr_j = trow_ref[:, pl.ds(j * _S, _S)]
                if j == k:
                    g = _ntb(fb_s, fb_s)
                    hb_j = hbr_s
                else:
                    g = _ntb(fb_s, fall_scr[pl.ds(j * _S, _S), :])
                    hb_j = hb_scr[:, pl.ds(j * _S, _S)]
                cm, rc = _block_updates(g, hb_s, hb_j, tc_k, tr_j,
                                        labels_row)
                if j == k:
                    colmax_scr[:, pl.ds(j * _S, _S)] = cm
                else:
                    colmax_scr[:, pl.ds(j * _S, _S)] = jnp.maximum(
                        colmax_scr[:, pl.ds(j * _S, _S)], cm)
                rcls_acc = rc if rcls_acc is None else jnp.maximum(rcls_acc, rc)
            if k == 0:
                rcls_scr[...] = rcls_acc
            else:
                rcls_scr[...] = jnp.maximum(rcls_scr[...], rcls_acc)

    @pl.when(s == _NS - 1)
    def _finalize():
        colmax_all = colmax_scr[...]
        rcls_all = rcls_scr[...]

        t_row = trow_ref[...]
        cmask = lax.broadcasted_iota(jnp.int32, (_C, 1), 0) == t_row
        ccls_col = jnp.max(jnp.where(cmask, colmax_all, _NEG_INF),
                           axis=1, keepdims=True)
        eye = (lax.broadcasted_iota(jnp.int32, (_C, 1), 0) ==
               lax.broadcasted_iota(jnp.int32, (1, _C), 1)).astype(jnp.float32)
        rcls_fin = jnp.maximum(rcls_all, -3.0e38)
        rcls_col = _nt(eye, rcls_fin)
        half_max = jnp.maximum(ccls_col, rcls_col)
        cmax = jnp.sqrt(jnp.clip(2.0 * half_max, 1e-12, None))
        counts_col = jnp.sum(cmask.astype(jnp.float32), axis=1,
                             keepdims=True)
        contrib = jnp.where(counts_col >= 2.0, 1.0 / cmax, 0.0)
        intra = jnp.sum(contrib)

        centers = cen_scr[...] / jnp.maximum(counts_col, 1.0)
        cc = centers * centers
        csq_col = jnp.sum(cc, axis=1, keepdims=True)
        csq_row = _nt(ones_row, cc)
        gc = _nt(centers, centers)
        dc = jnp.sqrt(jnp.clip(csq_col + csq_row - 2.0 * gc, 1e-12, None))
        t_col = tcol_ref[...]
        onehot_nc = (t_col == labels_row).astype(jnp.float32)
        counts_row = lax.dot_general(jnp.ones((1, _N), jnp.float32), onehot_nc,
                                     (((1,), (0,)), ((), ())),
                                     preferred_element_type=jnp.float32)
        valid = (counts_col > 0.0) & (counts_row > 0.0) & (dc > 0.0)
        min_inter = jnp.min(jnp.where(valid, dc, _POS_INF))

        out_ref[0, 0] = _ALPHA * (_MARGIN - min_inter) + _BETA * intra


def kernel(features, targets):
    t_col = targets.reshape(_N, 1).astype(jnp.int32)
    t_row = targets.reshape(1, _N).astype(jnp.int32)
    out = pl.pallas_call(
        _body,
        grid=(_NS,),
        in_specs=[
            pl.BlockSpec((_S, _D), lambda s: (s, 0)),
            pl.BlockSpec((_N, 1), lambda s: (0, 0)),
            pl.BlockSpec((1, _N), lambda s: (0, 0)),
        ],
        out_specs=pl.BlockSpec(memory_space=pltpu.SMEM),
        out_shape=jax.ShapeDtypeStruct((1, 1), jnp.float32),
        scratch_shapes=[
            pltpu.VMEM((_N, _D), jnp.bfloat16),
            pltpu.VMEM((1, _N), jnp.bfloat16),
            pltpu.VMEM((_C, _D), jnp.float32),
            pltpu.VMEM((1, _N), jnp.float32),
            pltpu.VMEM((1, _C), jnp.float32),
        ],
    )(features, t_col, t_row)
    return out[0, 0]


# speedup vs baseline: 1.0498x; 1.0498x over previous
import jax
import jax.numpy as jnp
from jax import lax
from jax.experimental import pallas as pl
from jax.experimental.pallas import tpu as pltpu

_MARGIN = 0.1
_ALPHA = 0.5
_BETA = 0.5
_C = 32
_N = 1024
_D = 2048
_S = _N // 2
_NEG_INF = float('-inf')
_POS_INF = float('inf')


def _nt(a, b):
    return lax.dot_general(a, b, (((1,), (1,)), ((), ())),
                           preferred_element_type=jnp.float32)


def _ntb(a, b):
    return lax.dot_general(a, b, (((1,), (1,)), ((), ())),
                           preferred_element_type=jnp.float32
                           ).astype(jnp.bfloat16)


def _block_updates(g, hsq_col, hsq_row, tc, tr, labels_row):
    v = jnp.where(tc == tr, hsq_col + hsq_row - g,
                  jnp.bfloat16(_NEG_INF))
    colmax = jnp.max(v, axis=0, keepdims=True).astype(jnp.float32)
    rowmax = jnp.max(v, axis=1, keepdims=True).astype(jnp.float32)
    rcls = jnp.max(jnp.where(tc == labels_row, rowmax, _NEG_INF),
                   axis=0, keepdims=True)
    return colmax, rcls


def _body(fs_ref, tcol_ref, trow_ref, out_ref,
          f0_scr, hsqr_scr, hb_scr, cen_scr, colmax_scr, rcls_scr):
    s = pl.program_id(0)
    fs = fs_ref[...]
    labels_row = lax.broadcasted_iota(jnp.int32, (1, _C), 1)

    ones_row = jnp.ones((1, _D), jnp.float32)
    ff = fs * fs
    hsq_s = 0.5 * _nt(ff, ones_row)
    hsqr_s = 0.5 * _nt(ones_row, ff)
    hsqr_scr[:, pl.ds(s * _S, _S)] = hsqr_s
    fb_s = fs.astype(jnp.bfloat16)
    hb_s = hsq_s.astype(jnp.bfloat16)
    hbr_s = hsqr_s.astype(jnp.bfloat16)

    tc_s = tcol_ref[pl.ds(s * _S, _S), :]
    onehot_s = (tc_s == labels_row).astype(jnp.float32)
    cen_part = lax.dot_general(onehot_s, fs, (((0,), (0,)), ((), ())),
                               preferred_element_type=jnp.float32)

    @pl.when(s == 0)
    def _step0():
        f0_scr[...] = fb_s
        hb_scr[...] = hbr_s
        cen_scr[...] = cen_part
        tr0 = trow_ref[:, pl.ds(0, _S)]
        g00 = _ntb(fb_s, fb_s)
        cm0, rc0 = _block_updates(g00, hb_s, hbr_s, tc_s, tr0, labels_row)
        colmax_scr[:, pl.ds(0, _S)] = cm0
        rcls_scr[...] = rc0

    @pl.when(s == 1)
    def _step1():
        cen = cen_scr[...] + cen_part
        f0 = f0_scr[...]
        tr0 = trow_ref[:, pl.ds(0, _S)]
        tr1 = trow_ref[:, pl.ds(_S, _S)]
        hb0 = hb_scr[...]

        g10 = _ntb(fb_s, f0)
        cm10, rc10 = _block_updates(g10, hb_s, hb0, tc_s, tr0, labels_row)
        g11 = _ntb(fb_s, fb_s)
        cm11, rc11 = _block_updates(g11, hb_s, hbr_s, tc_s, tr1, labels_row)

        colmax0 = jnp.maximum(colmax_scr[:, pl.ds(0, _S)], cm10)
        colmax_all = jnp.concatenate([colmax0, cm11], axis=1)
        rcls_all = jnp.maximum(rcls_scr[...], jnp.maximum(rc10, rc11))

        t_row = trow_ref[...]
        cmask = lax.broadcasted_iota(jnp.int32, (_C, 1), 0) == t_row
        ccls_col = jnp.max(jnp.where(cmask, colmax_all, _NEG_INF),
                           axis=1, keepdims=True)
        eye = (lax.broadcasted_iota(jnp.int32, (_C, 1), 0) ==
               lax.broadcasted_iota(jnp.int32, (1, _C), 1)).astype(jnp.float32)
        rcls_fin = jnp.maximum(rcls_all, -3.0e38)
        rcls_col = _nt(eye, rcls_fin)
        half_max = jnp.maximum(ccls_col, rcls_col)
        cmax = jnp.sqrt(jnp.clip(2.0 * half_max, 1e-12, None))
        counts_col = jnp.sum(cmask.astype(jnp.float32), axis=1,
                             keepdims=True)
        contrib = jnp.where(counts_col >= 2.0, 1.0 / cmax, 0.0)
        intra = jnp.sum(contrib)

        centers = cen / jnp.maximum(counts_col, 1.0)
        cc = centers * centers
        csq_col = jnp.sum(cc, axis=1, keepdims=True)
        csq_row = _nt(ones_row, cc)
        gc = _nt(centers, centers)
        dc = jnp.sqrt(jnp.clip(csq_col + csq_row - 2.0 * gc, 1e-12, None))
        t_col = tcol_ref[...]
        onehot_nc = (t_col == labels_row).astype(jnp.float32)
        counts_row = lax.dot_general(jnp.ones((1, _N), jnp.float32), onehot_nc,
                                     (((1,), (0,)), ((), ())),
                                     preferred_element_type=jnp.float32)
        valid = (counts_col > 0.0) & (counts_row > 0.0) & (dc > 0.0)
        min_inter = jnp.min(jnp.where(valid, dc, _POS_INF))

        out_ref[0, 0] = _ALPHA * (_MARGIN - min_inter) + _BETA * intra


def kernel(features, targets):
    t_col = targets.reshape(_N, 1).astype(jnp.int32)
    t_row = targets.reshape(1, _N).astype(jnp.int32)
    out = pl.pallas_call(
        _body,
        grid=(2,),
        in_specs=[
            pl.BlockSpec((_S, _D), lambda s: (s, 0)),
            pl.BlockSpec((_N, 1), lambda s: (0, 0)),
            pl.BlockSpec((1, _N), lambda s: (0, 0)),
        ],
        out_specs=pl.BlockSpec(memory_space=pltpu.SMEM),
        out_shape=jax.ShapeDtypeStruct((1, 1), jnp.float32),
        scratch_shapes=[
            pltpu.VMEM((_S, _D), jnp.bfloat16),
            pltpu.VMEM((1, _N), jnp.float32),
            pltpu.VMEM((1, _S), jnp.bfloat16),
            pltpu.VMEM((_C, _D), jnp.float32),
            pltpu.VMEM((1, _N), jnp.float32),
            pltpu.VMEM((1, _C), jnp.float32),
        ],
    )(features, t_col, t_row)
    return out[0, 0]
